# baseline (speedup 1.0000x reference)
import functools

import jax
import jax.numpy as jnp
from jax import lax
from jax.experimental import pallas as pl
from jax.experimental.pallas import tpu as pltpu
from jax.experimental.pallas import tpu_sc as plsc

VOCAB = 1000000
LSEQ = 200
D = 64
BATCH = 4096

NC = 2
NS = 16
NW = NC * NS

TOT = BATCH * LSEQ
RW = TOT // NW
S = 400
NCH = RW // S
G = 100
NG = S // G
IDXROWS = TOT // G


def _sc_body(tok_hbm, idx_hbm, pos2_hbm, out_hbm, idx_v, rows_v, pos_v,
             gsem, ssem):
    wid = lax.axis_index("s") * NC + lax.axis_index("c")
    base_row = wid * RW
    base_irow = wid * (RW // G)

    pltpu.sync_copy(pos2_hbm, pos_v)

    @pl.loop(0, NCH)
    def _chunk(c):
        irow = base_irow + c * NG
        pltpu.sync_copy(idx_hbm.at[pl.ds(irow, NG)], idx_v)
        descs = []
        for j in range(NG):
            descs.append(
                pltpu.async_copy(tok_hbm.at[idx_v.at[j]],
                                 rows_v.at[pl.ds(j * G, G)], gsem))
        for d in descs:
            d.wait()

        @pl.loop(0, S)
        def _row(r):
            for e in range(D // 16):
                sl = pl.ds(e * 16, 16)
                rows_v[r, sl] = rows_v[r, sl] + pos_v[r, sl]

        out_row = base_row + c * S
        pltpu.sync_copy(rows_v, out_hbm.at[pl.ds(out_row, S)])


@functools.partial(jax.jit, static_argnums=())
def _sc_embed(xf2, token_table, pos2):
    mesh = plsc.VectorSubcoreMesh(core_axis_name="c", subcore_axis_name="s")
    fn = pl.kernel(
        _sc_body,
        out_type=jax.ShapeDtypeStruct((TOT, D), jnp.float32),
        mesh=mesh,
        scratch_types=[
            pltpu.VMEM((NG, G), jnp.int32),
            pltpu.VMEM((S, D), jnp.float32),
            pltpu.VMEM((S, D), jnp.float32),
            pltpu.SemaphoreType.DMA,
            pltpu.SemaphoreType.DMA,
        ],
        compiler_params=pltpu.CompilerParams(use_tc_tiling_on_sc=False),
    )
    return fn(token_table, xf2, pos2)


def kernel(x, token_table, pos_table):
    xf2 = x.reshape(IDXROWS, G).astype(jnp.int32)
    pos2 = jnp.concatenate([pos_table, pos_table], axis=0)
    out = _sc_embed(xf2, token_table, pos2)
    return out.reshape(BATCH, LSEQ, D)
